# baseline (speedup 1.0000x reference)
import functools

import jax
import jax.numpy as jnp
from jax import lax
from jax.experimental import pallas as pl
from jax.experimental.pallas import tpu as pltpu
from jax.experimental.pallas import tpu_sc as plsc

N = 10000
E = 320000
D = 128
ALPHA1 = 0.005
ALPHA2 = 1.0
BETA1 = 1.0
BETA2 = 0.1
EPS = 1e-15

_INFO = plsc.get_sparse_core_info()
NC = _INFO.num_cores
NS = _INFO.num_subcores
L = _INFO.num_lanes
NW = NC * NS
EW = E // NW
NV = EW // L
GROUP = 25
NG = NV // GROUP
NG0 = 20
CW0 = NG0 * GROUP * L

assert E % NW == 0 and EW % L == 0 and NV % GROUP == 0

_PIB = jax.lax.GatherScatterMode.PROMISE_IN_BOUNDS
_GDN = lax.GatherDimensionNumbers(
    offset_dims=(), collapsed_slice_dims=(0,), start_index_map=(0,))


def _splat(vec, j):
    idx = jnp.full((L, 1), j, jnp.int32)
    return lax.gather(vec, idx, _GDN, slice_sizes=(1,), mode=_PIB)


def _sc_scan_kernel(ei_hbm, em_hbm, feat_hbm, xy_hbm,
                    out_hbm, dstv, xyv, s16, e16, rows, accv,
                    dsem, dsem2, sem):
    wid = lax.axis_index("s") * NC + lax.axis_index("c")
    base = wid * EW
    dcp = pltpu.async_copy(ei_hbm.at[1, pl.ds(base, CW0)],
                           dstv.at[pl.ds(0, CW0)], dsem)
    dcp2 = pltpu.async_copy(ei_hbm.at[1, pl.ds(base + CW0, EW - CW0)],
                            dstv.at[pl.ds(CW0, EW - CW0)], dsem2)
    pltpu.sync_copy(xy_hbm, xyv.at[pl.ds(0, 2)])
    v = xyv[...]
    xvec = _splat(v, 0)
    yvec = _splat(v, 1)
    thr = jnp.max(jnp.maximum(xvec, yvec))

    for k in range(4):
        for b in range(D // L):
            accv[k, pl.ds(b * L, L)] = jnp.zeros((L,), jnp.float32)

    def handle_vec(off):
        scp = pltpu.async_copy(ei_hbm.at[0, pl.ds(base + off, L)], s16, dsem)
        ecp = pltpu.async_copy(em_hbm.at[pl.ds(base + off, L)], e16, sem)
        scp.wait()
        rcp = pltpu.async_copy(feat_hbm.at[s16], rows, dsem)
        ecp.wait()
        rcp.wait()
        d16 = dstv[pl.ds(off, L)]
        e16v = e16[...]
        wfx = (d16 == xvec).astype(jnp.float32)
        wfy = (d16 == yvec).astype(jnp.float32)
        kp = (e16v >= 0.0).astype(jnp.float32)
        wmx = wfx * kp
        wmy = wfy * kp
        parts = [[jnp.zeros((L,), jnp.float32) for _ in range(D // L)]
                 for _ in range(4)]
        for j in range(L):
            ws = (_splat(wfx, j), _splat(wfy, j), _splat(wmx, j), _splat(wmy, j))
            for b in range(D // L):
                row_v = rows[j, pl.ds(b * L, L)]
                for k in range(4):
                    parts[k][b] = parts[k][b] + ws[k] * row_v
        for k in range(4):
            for b in range(D // L):
                accv[k, pl.ds(b * L, L)] = (accv[k, pl.ds(b * L, L)]
                                            + parts[k][b])

    def fine_body(j, goff):
        off = goff + j * L
        d16 = dstv[pl.ds(off, L)]
        hit = (d16 == xvec) | (d16 == yvec)
        cnt = jnp.max(hit.astype(jnp.int32))

        @pl.when(cnt > 0)
        def _():
            handle_vec(off)

        return goff

    def group_body(g, _):
        goff = g * (GROUP * L)
        mn = dstv[pl.ds(goff, L)]
        for j in range(1, GROUP):
            mn = jnp.minimum(mn, dstv[pl.ds(goff + j * L, L)])
        gmin = jnp.min(mn)

        @pl.when(gmin <= thr)
        def _():
            lax.fori_loop(0, GROUP, fine_body, goff)

        return 0

    dcp.wait()
    lax.fori_loop(0, NG0, group_body, 0)
    dcp2.wait()
    lax.fori_loop(NG0, NG, group_body, 0)
    pltpu.sync_copy(accv, out_hbm.at[wid])


def _sc_scan(ei, em, feat, xy):
    mesh = plsc.VectorSubcoreMesh(core_axis_name="c", subcore_axis_name="s")
    f = functools.partial(
        pl.kernel,
        mesh=mesh,
        compiler_params=pltpu.CompilerParams(
            needs_layout_passes=False, use_tc_tiling_on_sc=False),
        out_type=jax.ShapeDtypeStruct((NW, 4, D), jnp.float32),
        scratch_types=[
            pltpu.VMEM((EW,), jnp.int32),
            pltpu.VMEM((L,), jnp.int32),
            pltpu.VMEM((L,), jnp.int32),
            pltpu.VMEM((L,), jnp.float32),
            pltpu.VMEM((L, D), jnp.float32),
            pltpu.VMEM((4, D), jnp.float32),
            pltpu.SemaphoreType.DMA,
            pltpu.SemaphoreType.DMA,
            pltpu.SemaphoreType.DMA,
        ],
    )(_sc_scan_kernel)
    return f(ei, em, feat, xy)


def _tc_em_kernel(em_ref, out_ref):
    m = em_ref[...]
    em = jax.nn.sigmoid(m)
    ent_e = -em * jnp.log(em + EPS) - (1.0 - em) * jnp.log(1.0 - em + EPS)
    out_ref[0, 0] = jnp.sum(em)
    out_ref[0, 1] = jnp.sum(ent_e)


def _tc_em(edge_mask):
    return pl.pallas_call(
        _tc_em_kernel,
        out_shape=jax.ShapeDtypeStruct((1, 2), jnp.float32),
        out_specs=pl.BlockSpec(memory_space=pltpu.SMEM),
    )(edge_mask)


def _tc_combine_kernel(sums_ref, fm_ref, w_ref, part_ref, out_ref):
    fm = jax.nn.sigmoid(fm_ref[...])
    ent_f = -fm * jnp.log(fm + EPS) - (1.0 - fm) * jnp.log(1.0 - fm + EPS)
    s = jnp.sum(part_ref[...], axis=0)
    rsel = (lax.broadcasted_iota(jnp.int32, (4, 1), 0) >= 2).astype(jnp.float32)
    scale = 1.0 + rsel * (fm - 1.0)
    logits = jnp.dot(s * scale, w_ref[...],
                     preferred_element_type=jnp.float32)
    pred = jnp.sum(logits[0:1] * logits[1:2])
    lp = jnp.sum(logits[2:3] * logits[3:4])
    loss = (lp - pred
            + ALPHA1 * sums_ref[0, 0]
            + ALPHA2 * (sums_ref[0, 1] / E)
            + BETA1 * jnp.mean(fm)
            + BETA2 * jnp.mean(ent_f))
    out_ref[0, 0] = loss


def _tc_combine(sums, feat_mask, W, partials):
    return pl.pallas_call(
        _tc_combine_kernel,
        out_shape=jax.ShapeDtypeStruct((1, 1), jnp.float32),
        in_specs=[
            pl.BlockSpec(memory_space=pltpu.SMEM),
            pl.BlockSpec(memory_space=pltpu.VMEM),
            pl.BlockSpec(memory_space=pltpu.VMEM),
            pl.BlockSpec(memory_space=pltpu.VMEM),
        ],
        out_specs=pl.BlockSpec(memory_space=pltpu.SMEM),
    )(sums, feat_mask, W, partials)


def kernel(feat, feat_mask, edge_mask, W, edge_index, x, y):
    xy = jnp.stack([jnp.asarray(x, jnp.int32), jnp.asarray(y, jnp.int32)])
    partials = _sc_scan(edge_index, edge_mask, feat, xy)
    sums = _tc_em(edge_mask)
    loss = _tc_combine(sums, feat_mask, W, partials)
    return loss[0, 0]


# speedup vs baseline: 1.0138x; 1.0138x over previous
import functools

import jax
import jax.numpy as jnp
from jax import lax
from jax.experimental import pallas as pl
from jax.experimental.pallas import tpu as pltpu
from jax.experimental.pallas import tpu_sc as plsc

N = 10000
E = 320000
D = 128
ALPHA1 = 0.005
ALPHA2 = 1.0
BETA1 = 1.0
BETA2 = 0.1
EPS = 1e-15

_INFO = plsc.get_sparse_core_info()
NC = _INFO.num_cores
NS = _INFO.num_subcores
L = _INFO.num_lanes
NW = NC * NS
EW = E // NW
NV = EW // L
GROUP = 25
NG = NV // GROUP

assert E % NW == 0 and EW % L == 0 and NV % GROUP == 0

_PIB = jax.lax.GatherScatterMode.PROMISE_IN_BOUNDS
_GDN = lax.GatherDimensionNumbers(
    offset_dims=(), collapsed_slice_dims=(0,), start_index_map=(0,))


def _splat(vec, j):
    idx = jnp.full((L, 1), j, jnp.int32)
    return lax.gather(vec, idx, _GDN, slice_sizes=(1,), mode=_PIB)


def _sc_scan_kernel(ei_hbm, em_hbm, feat_hbm, xy_hbm,
                    out_hbm, dstv, xyv, s16, e16, rows, accv,
                    dsem, sem):
    wid = lax.axis_index("s") * NC + lax.axis_index("c")
    base = wid * EW
    dcp = pltpu.async_copy(ei_hbm.at[1, pl.ds(base, EW)], dstv, dsem)
    pltpu.sync_copy(xy_hbm, xyv.at[pl.ds(0, 2)])
    v = xyv[...]
    xvec = _splat(v, 0)
    yvec = _splat(v, 1)
    thr = jnp.max(jnp.maximum(xvec, yvec))

    for k in range(4):
        for b in range(D // L):
            accv[k, pl.ds(b * L, L)] = jnp.zeros((L,), jnp.float32)

    def handle_vec(off):
        scp = pltpu.async_copy(ei_hbm.at[0, pl.ds(base + off, L)], s16, dsem)
        ecp = pltpu.async_copy(em_hbm.at[pl.ds(base + off, L)], e16, sem)
        scp.wait()
        rcp = pltpu.async_copy(feat_hbm.at[s16], rows, dsem)
        ecp.wait()
        rcp.wait()
        d16 = dstv[pl.ds(off, L)]
        e16v = e16[...]
        wfx = (d16 == xvec).astype(jnp.float32)
        wfy = (d16 == yvec).astype(jnp.float32)
        kp = (e16v >= 0.0).astype(jnp.float32)
        wmx = wfx * kp
        wmy = wfy * kp
        parts = [[jnp.zeros((L,), jnp.float32) for _ in range(D // L)]
                 for _ in range(4)]
        for j in range(L):
            ws = (_splat(wfx, j), _splat(wfy, j), _splat(wmx, j), _splat(wmy, j))
            for b in range(D // L):
                row_v = rows[j, pl.ds(b * L, L)]
                for k in range(4):
                    parts[k][b] = parts[k][b] + ws[k] * row_v
        for k in range(4):
            for b in range(D // L):
                accv[k, pl.ds(b * L, L)] = (accv[k, pl.ds(b * L, L)]
                                            + parts[k][b])

    def fine_body(j, goff):
        off = goff + j * L
        d16 = dstv[pl.ds(off, L)]
        hit = (d16 == xvec) | (d16 == yvec)
        cnt = jnp.max(hit.astype(jnp.int32))

        @pl.when(cnt > 0)
        def _():
            handle_vec(off)

        return goff

    def group_body(g, _):
        goff = g * (GROUP * L)
        mn = dstv[pl.ds(goff, L)]
        for j in range(1, GROUP):
            mn = jnp.minimum(mn, dstv[pl.ds(goff + j * L, L)])
        gmin = jnp.min(mn)

        @pl.when(gmin <= thr)
        def _():
            lax.fori_loop(0, GROUP, fine_body, goff)

        return 0

    dcp.wait()
    lax.fori_loop(0, NG, group_body, 0)
    pltpu.sync_copy(accv, out_hbm.at[wid])


def _sc_scan(ei, em, feat, xy):
    mesh = plsc.VectorSubcoreMesh(core_axis_name="c", subcore_axis_name="s")
    f = functools.partial(
        pl.kernel,
        mesh=mesh,
        compiler_params=pltpu.CompilerParams(
            needs_layout_passes=False, use_tc_tiling_on_sc=False),
        out_type=jax.ShapeDtypeStruct((NW, 4, D), jnp.float32),
        scratch_types=[
            pltpu.VMEM((EW,), jnp.int32),
            pltpu.VMEM((L,), jnp.int32),
            pltpu.VMEM((L,), jnp.int32),
            pltpu.VMEM((L,), jnp.float32),
            pltpu.VMEM((L, D), jnp.float32),
            pltpu.VMEM((4, D), jnp.float32),
            pltpu.SemaphoreType.DMA,
            pltpu.SemaphoreType.DMA,
        ],
    )(_sc_scan_kernel)
    return f(ei, em, feat, xy)


def _tc_em_kernel(em_ref, out_ref):
    m = em_ref[...]
    em = jax.nn.sigmoid(m)
    ent_e = -em * jnp.log(em + EPS) - (1.0 - em) * jnp.log(1.0 - em + EPS)
    out_ref[0, 0] = jnp.sum(em)
    out_ref[0, 1] = jnp.sum(ent_e)


def _tc_em(edge_mask):
    return pl.pallas_call(
        _tc_em_kernel,
        out_shape=jax.ShapeDtypeStruct((1, 2), jnp.float32),
        out_specs=pl.BlockSpec(memory_space=pltpu.SMEM),
    )(edge_mask)


def _tc_combine_kernel(sums_ref, fm_ref, w_ref, part_ref, out_ref):
    fm = jax.nn.sigmoid(fm_ref[...])
    ent_f = -fm * jnp.log(fm + EPS) - (1.0 - fm) * jnp.log(1.0 - fm + EPS)
    s = jnp.sum(part_ref[...], axis=0)
    rsel = (lax.broadcasted_iota(jnp.int32, (4, 1), 0) >= 2).astype(jnp.float32)
    scale = 1.0 + rsel * (fm - 1.0)
    logits = jnp.dot(s * scale, w_ref[...],
                     preferred_element_type=jnp.float32)
    pred = jnp.sum(logits[0:1] * logits[1:2])
    lp = jnp.sum(logits[2:3] * logits[3:4])
    loss = (lp - pred
            + ALPHA1 * sums_ref[0, 0]
            + ALPHA2 * (sums_ref[0, 1] / E)
            + BETA1 * jnp.mean(fm)
            + BETA2 * jnp.mean(ent_f))
    out_ref[0, 0] = loss


def _tc_combine(sums, feat_mask, W, partials):
    return pl.pallas_call(
        _tc_combine_kernel,
        out_shape=jax.ShapeDtypeStruct((1, 1), jnp.float32),
        in_specs=[
            pl.BlockSpec(memory_space=pltpu.SMEM),
            pl.BlockSpec(memory_space=pltpu.VMEM),
            pl.BlockSpec(memory_space=pltpu.VMEM),
            pl.BlockSpec(memory_space=pltpu.VMEM),
        ],
        out_specs=pl.BlockSpec(memory_space=pltpu.SMEM),
    )(sums, feat_mask, W, partials)


def kernel(feat, feat_mask, edge_mask, W, edge_index, x, y):
    xy = jnp.stack([jnp.asarray(x, jnp.int32), jnp.asarray(y, jnp.int32)])
    partials = _sc_scan(edge_index, edge_mask, feat, xy)
    sums = _tc_em(edge_mask)
    loss = _tc_combine(sums, feat_mask, W, partials)
    return loss[0, 0]
